# baseline (speedup 1.0000x reference)
import functools

import jax
import jax.numpy as jnp
import numpy as np
from jax import lax
from jax.experimental import pallas as pl

_ROWS = 128
_COLS = 32768
_N_KEEP = 1024
_CBLOCK = 16384
_WORDS = _ROWS // 32


@functools.lru_cache(maxsize=1)
def _keep_mask_words() -> np.ndarray:
    with jax.ensure_compile_time_eval():
        key = jax.random.key(42)
        r = jax.random.uniform(key, (_ROWS, _COLS), dtype=jnp.float32)
        perm = jnp.argsort(r, axis=-1)
        mask = perm < _N_KEEP
    m = np.asarray(mask, dtype=np.uint32).reshape(_WORDS, 32, _COLS)
    shifts = np.arange(32, dtype=np.uint32)[None, :, None]
    return (m << shifts).sum(axis=1, dtype=np.uint32)


def _select_kernel(x_ref, w_ref, o_ref):
    shift = lax.broadcasted_iota(jnp.uint32, (32, _CBLOCK), 0)
    for k in range(_WORDS):
        x = x_ref[32 * k : 32 * (k + 1), :]
        bits = jnp.broadcast_to(w_ref[k : k + 1, :], (32, _CBLOCK))
        m = ((bits >> shift) & 1) != 0
        y = (x > 0.5).astype(jnp.float32)
        o_ref[32 * k : 32 * (k + 1), :] = jnp.where(m, y, x)


def kernel(input):
    w = _keep_mask_words()
    return pl.pallas_call(
        _select_kernel,
        out_shape=jax.ShapeDtypeStruct((_ROWS, _COLS), jnp.float32),
        grid=(_COLS // _CBLOCK,),
        in_specs=[
            pl.BlockSpec((_ROWS, _CBLOCK), lambda i: (0, i)),
            pl.BlockSpec((_WORDS, _CBLOCK), lambda i: (0, i)),
        ],
        out_specs=pl.BlockSpec((_ROWS, _CBLOCK), lambda i: (0, i)),
    )(input, w)


# speedup vs baseline: 1.0108x; 1.0108x over previous
import functools

import jax
import jax.numpy as jnp
import numpy as np
from jax import lax
from jax.experimental import pallas as pl

_ROWS = 128
_COLS = 32768
_N_KEEP = 1024
_CBLOCK = 16384
_WORDS = _ROWS // 32


@functools.lru_cache(maxsize=1)
def _keep_mask_words() -> np.ndarray:
    with jax.ensure_compile_time_eval():
        key = jax.random.key(42)
        r = jax.random.uniform(key, (_ROWS, _COLS), dtype=jnp.float32)
        perm = jnp.argsort(r, axis=-1)
        mask = perm < _N_KEEP
    m = np.asarray(mask, dtype=np.uint32).reshape(_WORDS, 32, _COLS)
    shifts = np.arange(32, dtype=np.uint32)[None, :, None]
    return (m << shifts).sum(axis=1, dtype=np.uint32)


def _select_kernel(x_ref, w_ref, o_ref):
    onehot = jnp.uint32(1) << lax.broadcasted_iota(jnp.uint32, (32, _CBLOCK), 0)
    for k in range(_WORDS):
        x = x_ref[32 * k : 32 * (k + 1), :]
        bits = jnp.broadcast_to(w_ref[k : k + 1, :], (32, _CBLOCK))
        m = (bits & onehot) != 0
        y = (x > 0.5).astype(jnp.float32)
        o_ref[32 * k : 32 * (k + 1), :] = jnp.where(m, y, x)


def kernel(input):
    w = _keep_mask_words()
    return pl.pallas_call(
        _select_kernel,
        out_shape=jax.ShapeDtypeStruct((_ROWS, _COLS), jnp.float32),
        grid=(_COLS // _CBLOCK,),
        in_specs=[
            pl.BlockSpec((_ROWS, _CBLOCK), lambda i: (0, i)),
            pl.BlockSpec((_WORDS, _CBLOCK), lambda i: (0, i)),
        ],
        out_specs=pl.BlockSpec((_ROWS, _CBLOCK), lambda i: (0, i)),
    )(input, w)
